# Initial kernel scaffold; baseline (speedup 1.0000x reference)
import jax
import jax.numpy as jnp
from jax.experimental import pallas as pl


def kernel(tokenized_prompts, ctx, token_embedding):
    raise NotImplementedError("write your pallas kernel here")


# speedup vs baseline: 1.8821x; 1.8821x over previous
import functools

import jax
import jax.numpy as jnp
from jax import lax
from jax.experimental import pallas as pl
from jax.experimental.pallas import tpu as pltpu
from jax.experimental.pallas import tpu_sc as plsc


def kernel(tokenized_prompts, ctx, token_embedding):
    n_cls, seq = tokenized_prompts.shape
    n_ctx, d = ctx.shape
    n_suf = seq - 1 - n_ctx

    info = plsc.get_sparse_core_info()
    nc, ns = info.num_cores, info.num_subcores
    nw = nc * ns
    per_w = n_cls // nw

    mesh = plsc.VectorSubcoreMesh(core_axis_name="c", subcore_axis_name="s")

    @functools.partial(
        pl.kernel,
        out_type=jax.ShapeDtypeStruct((n_cls, seq, d), jnp.float32),
        mesh=mesh,
        scratch_types=[
            pltpu.VMEM((per_w, seq), jnp.int32),
            pltpu.VMEM((2, seq, d), jnp.float32),
            pltpu.VMEM((n_ctx, d), jnp.float32),
            pltpu.SemaphoreType.DMA,
            pltpu.SemaphoreType.DMA,
            pltpu.SemaphoreType.DMA,
            pltpu.SemaphoreType.DMA,
        ],
        compiler_params=pltpu.CompilerParams(use_tc_tiling_on_sc=False),
    )
    def _gather_kernel(tok_hbm, ctx_hbm, table_hbm, out_hbm,
                       idx_v, rows_v, ctx_v, g0, g1, s0, s1):
        wid = lax.axis_index("s") * nc + lax.axis_index("c")
        base = wid * per_w
        gsems = (g0, g1)
        ssems = (s0, s1)

        pltpu.sync_copy(ctx_hbm, ctx_v)
        pltpu.sync_copy(tok_hbm.at[pl.ds(base, per_w)], idx_v)

        pltpu.async_copy(table_hbm.at[idx_v.at[0]], rows_v.at[0], g0)
        pltpu.async_copy(table_hbm.at[idx_v.at[1]], rows_v.at[1], g1)

        @pl.loop(0, per_w, step=2)
        def _body(n):
            for b in (0, 1):
                m = n + b
                pltpu.make_async_copy(
                    table_hbm.at[idx_v.at[m]], rows_v.at[b], gsems[b]
                ).wait()
                row = out_hbm.at[base + m]
                pltpu.async_copy(
                    rows_v.at[b, pl.ds(0, 1)], row.at[pl.ds(0, 1)], ssems[b])
                pltpu.async_copy(ctx_v, row.at[pl.ds(1, n_ctx)], ssems[b])
                pltpu.async_copy(
                    rows_v.at[b, pl.ds(1 + n_ctx, n_suf)],
                    row.at[pl.ds(1 + n_ctx, n_suf)], ssems[b])
                pltpu.make_async_copy(
                    rows_v.at[b, pl.ds(0, 1)], row.at[pl.ds(0, 1)], ssems[b]
                ).wait()
                pltpu.make_async_copy(
                    ctx_v, row.at[pl.ds(1, n_ctx)], ssems[b]).wait()
                pltpu.make_async_copy(
                    rows_v.at[b, pl.ds(1 + n_ctx, n_suf)],
                    row.at[pl.ds(1 + n_ctx, n_suf)], ssems[b]).wait()

                @pl.when(m + 2 < per_w)
                def _():
                    pltpu.async_copy(
                        table_hbm.at[idx_v.at[m + 2]], rows_v.at[b], gsems[b]
                    )

    return _gather_kernel(tokenized_prompts, ctx, token_embedding)


# speedup vs baseline: 1.8883x; 1.0033x over previous
import functools

import jax
import jax.numpy as jnp
from jax import lax
from jax.experimental import pallas as pl
from jax.experimental.pallas import tpu as pltpu
from jax.experimental.pallas import tpu_sc as plsc


def kernel(tokenized_prompts, ctx, token_embedding):
    n_cls, seq = tokenized_prompts.shape
    n_ctx, d = ctx.shape
    n_suf = seq - 1 - n_ctx
    ng = 1 + n_suf

    tok_c = jnp.concatenate(
        [tokenized_prompts[:, :1], tokenized_prompts[:, 1 + n_ctx:]], axis=1)

    info = plsc.get_sparse_core_info()
    nc, ns = info.num_cores, info.num_subcores
    nw = nc * ns
    per_w = n_cls // nw
    nbuf = 3

    mesh = plsc.VectorSubcoreMesh(core_axis_name="c", subcore_axis_name="s")

    @functools.partial(
        pl.kernel,
        out_type=jax.ShapeDtypeStruct((n_cls, seq, d), jnp.float32),
        mesh=mesh,
        scratch_types=[
            pltpu.VMEM((per_w, ng), jnp.int32),
            pltpu.VMEM((nbuf, ng, d), jnp.float32),
            pltpu.VMEM((n_ctx, d), jnp.float32),
            [pltpu.SemaphoreType.DMA] * nbuf,
            [pltpu.SemaphoreType.DMA] * nbuf,
        ],
        compiler_params=pltpu.CompilerParams(use_tc_tiling_on_sc=False),
    )
    def _gather_kernel(tok_hbm, ctx_hbm, table_hbm, out_hbm,
                       idx_v, rows_v, ctx_v, gsems, ssems):
        wid = lax.axis_index("s") * nc + lax.axis_index("c")
        base = wid * per_w

        pltpu.sync_copy(ctx_hbm, ctx_v)
        pltpu.sync_copy(tok_hbm.at[pl.ds(base, per_w)], idx_v)

        def issue_gather(k, b):
            pltpu.async_copy(
                table_hbm.at[idx_v.at[k]], rows_v.at[b], gsems[b])

        def store_descs(k, b):
            row = out_hbm.at[base + k]
            return (
                (rows_v.at[b, pl.ds(0, 1)], row.at[pl.ds(0, 1)]),
                (ctx_v, row.at[pl.ds(1, n_ctx)]),
                (rows_v.at[b, pl.ds(1, n_suf)], row.at[pl.ds(1 + n_ctx, n_suf)]),
            )

        issue_gather(0, 0)

        @pl.loop(0, per_w, step=nbuf)
        def _body(n):
            for b in range(nbuf):
                k = n + b
                bn = (b + 1) % nbuf

                @pl.when(k < per_w)
                def _():
                    pltpu.make_async_copy(
                        table_hbm.at[idx_v.at[k]], rows_v.at[b], gsems[b]
                    ).wait()
                    for src, dst in store_descs(k, b):
                        pltpu.async_copy(src, dst, ssems[b])

                @pl.when(k >= 2)
                def _():
                    for src, dst in store_descs(k - 2, bn):
                        pltpu.make_async_copy(src, dst, ssems[bn]).wait()

                @pl.when(k + 1 < per_w)
                def _():
                    issue_gather(k + 1, bn)

        for k in range(max(0, per_w - 1), per_w):
            b = k % nbuf
            for src, dst in store_descs(k, b):
                pltpu.make_async_copy(src, dst, ssems[b]).wait()

    return _gather_kernel(tok_c, ctx, token_embedding)
